# baseline (speedup 1.0000x reference)
import jax
import jax.numpy as jnp
from jax.experimental import pallas as pl

EMBED = 768
HEADS = 12
HD = EMBED // HEADS
WS = 16
SEQ = 2048
NW = SEQ // WS
TOPK = 4
SCALE = HD ** -0.5
BQ = 2048
BF = jnp.bfloat16
F32 = jnp.float32


def _proj_h_kernel(out_dtype, x_ref, w_ref, b_ref, o_ref):
    o = jax.lax.dot_general(x_ref[...].astype(BF), w_ref[0].astype(BF),
                            (((1,), (1,)), ((), ())),
                            preferred_element_type=F32) + b_ref[0]
    o_ref[0] = o.astype(out_dtype)


def _proj_h(x, w, b, out_dtype):
    import functools
    return pl.pallas_call(
        functools.partial(_proj_h_kernel, out_dtype),
        grid=(SEQ // BQ, HEADS),
        in_specs=[
            pl.BlockSpec((BQ, EMBED), lambda i, h: (i, 0)),
            pl.BlockSpec((1, HD, EMBED), lambda i, h: (h, 0, 0)),
            pl.BlockSpec((1, 1, HD), lambda i, h: (h, 0, 0)),
        ],
        out_specs=pl.BlockSpec((1, BQ, HD), lambda i, h: (h, i, 0)),
        out_shape=jax.ShapeDtypeStruct((HEADS, SEQ, HD), out_dtype),
    )(x, w.reshape(HEADS, HD, EMBED), b.reshape(HEADS, 1, HD))


def _proj_kT_kernel(x_ref, w_ref, b_ref, of_ref, ob_ref):
    o = jax.lax.dot_general(w_ref[0].astype(BF), x_ref[...].astype(BF),
                            (((1,), (1,)), ((), ())),
                            preferred_element_type=F32) + b_ref[0]
    of_ref[0] = o
    ob_ref[0] = o.astype(BF)


def _proj_kT(x, w, b):
    return pl.pallas_call(
        _proj_kT_kernel,
        grid=(SEQ // BQ, HEADS),
        in_specs=[
            pl.BlockSpec((BQ, EMBED), lambda i, h: (i, 0)),
            pl.BlockSpec((1, HD, EMBED), lambda i, h: (h, 0, 0)),
            pl.BlockSpec((1, HD, 1), lambda i, h: (h, 0, 0)),
        ],
        out_specs=[
            pl.BlockSpec((1, HD, BQ), lambda i, h: (h, 0, i)),
            pl.BlockSpec((1, HD, BQ), lambda i, h: (h, 0, i)),
        ],
        out_shape=[
            jax.ShapeDtypeStruct((HEADS, HD, SEQ), F32),
            jax.ShapeDtypeStruct((HEADS, HD, SEQ), BF),
        ],
    )(x, w.reshape(HEADS, HD, EMBED), b.reshape(HEADS, HD, 1))


def _ens_kernel(k_ref, o_ref):
    kT = k_ref[0]
    nrm2 = jnp.sum(kT * kT, axis=0, keepdims=True)
    knT = kT / jnp.maximum(jnp.sqrt(nrm2), 1e-12)
    knb = knT.astype(BF).astype(F32)
    acc = jnp.zeros((HD * HD, NW), F32)
    for w in range(WS):
        a = knb[:, w * NW:(w + 1) * NW]
        rep = jnp.broadcast_to(a[:, None, :], (HD, HD, NW)).reshape(HD * HD, NW)
        til = jnp.broadcast_to(a[None, :, :], (HD, HD, NW)).reshape(HD * HD, NW)
        acc = acc + rep * til
    o_ref[0] = (acc * (1.0 / WS)).astype(BF)


def _ens(kT):
    return pl.pallas_call(
        _ens_kernel,
        grid=(HEADS,),
        in_specs=[pl.BlockSpec((1, HD, SEQ), lambda h: (h, 0, 0))],
        out_specs=pl.BlockSpec((1, HD * HD, NW), lambda h: (h, 0, 0)),
        out_shape=jax.ShapeDtypeStruct((HEADS, HD * HD, NW), BF),
    )(kT)


def _attn_kernel(t_ref, qt_ref, e_ref, kt_ref, v_ref, o_ref):
    qT = qt_ref[0]
    ensT = e_ref[0]
    kbT = kt_ref[0]
    vb = v_ref[0]
    t = jnp.maximum(t_ref[0, 0], 0.1)

    qn2 = jnp.sum(qT * qT, axis=0, keepdims=True)
    qnT = qT / jnp.maximum(jnp.sqrt(qn2), 1e-12)
    rep = jnp.broadcast_to(qnT[:, None, :], (HD, HD, BQ)).reshape(HD * HD, BQ)
    til = jnp.broadcast_to(qnT[None, :, :], (HD, HD, BQ)).reshape(HD * HD, BQ)
    qdmTb = (rep * til).astype(BF)
    score = jax.lax.dot_general(qdmTb, ensT, (((0,), (0,)), ((), ())),
                                preferred_element_type=F32) / t

    idx = jax.lax.broadcasted_iota(jnp.int32, (BQ, NW), 1)
    vals = []
    hots = []
    work = score
    for _ in range(TOPK):
        m = jnp.max(work, axis=1, keepdims=True)
        am = jnp.min(jnp.where(work == m, idx, NW), axis=1, keepdims=True)
        hot = idx == am
        vals.append(m)
        hots.append(hot)
        work = jnp.where(hot, jnp.float32(-1e30), work)
    es = [jnp.exp(v - vals[0]) for v in vals]
    z = es[0] + es[1] + es[2] + es[3]
    c = (
        es[0] * hots[0].astype(F32)
        + es[1] * hots[1].astype(F32)
        + es[2] * hots[2].astype(F32)
        + es[3] * hots[3].astype(F32)
    ) / z

    sf = jax.lax.dot_general(qT.astype(BF), kbT, (((0,), (0,)), ((), ())),
                             preferred_element_type=F32) * SCALE
    p3 = sf.reshape(BQ, WS, NW)
    pm = jnp.max(p3, axis=1, keepdims=True)
    e = jnp.exp(p3 - pm)
    se = jnp.sum(e, axis=1, keepdims=True)
    fa = e / se

    wts = fa.astype(BF).astype(F32) * c.astype(BF).astype(F32)[:, None, :]
    wb = wts.reshape(BQ, SEQ).astype(BF)
    o_ref[0] = jnp.dot(wb, vb, preferred_element_type=F32)


def _attention(temp2d, qhT, ensT, kbT, vbw):
    return pl.pallas_call(
        _attn_kernel,
        grid=(HEADS, SEQ // BQ),
        in_specs=[
            pl.BlockSpec((1, 1), lambda h, i: (0, 0)),
            pl.BlockSpec((1, HD, BQ), lambda h, i: (h, 0, i)),
            pl.BlockSpec((1, HD * HD, NW), lambda h, i: (h, 0, 0)),
            pl.BlockSpec((1, HD, SEQ), lambda h, i: (h, 0, 0)),
            pl.BlockSpec((1, SEQ, HD), lambda h, i: (h, 0, 0)),
        ],
        out_specs=pl.BlockSpec((1, BQ, HD), lambda h, i: (h, i, 0)),
        out_shape=jax.ShapeDtypeStruct((HEADS, SEQ, HD), F32),
    )(temp2d, qhT, ensT, kbT, vbw)


def _proj_out_kernel(x_ref, w_ref, b_ref, o_ref):
    acc = b_ref[...]
    for h in range(HEADS):
        acc = acc + jax.lax.dot_general(x_ref[h].astype(BF), w_ref[h],
                                        (((1,), (1,)), ((), ())),
                                        preferred_element_type=F32)
    o_ref[...] = acc


def _proj_out(attn3, wob, b):
    return pl.pallas_call(
        _proj_out_kernel,
        grid=(SEQ // BQ,),
        in_specs=[
            pl.BlockSpec((HEADS, BQ, HD), lambda i: (0, i, 0)),
            pl.BlockSpec((HEADS, EMBED, HD), lambda i: (0, 0, 0)),
            pl.BlockSpec((1, EMBED), lambda i: (0, 0)),
        ],
        out_specs=pl.BlockSpec((BQ, EMBED), lambda i: (i, 0)),
        out_shape=jax.ShapeDtypeStruct((SEQ, EMBED), F32),
    )(attn3, wob, b.reshape(1, EMBED))


@jax.jit
def kernel(query, key, value, Wq, bq, Wk, bk, Wv, bv, Wo, bo, temp):
    key_w = key[0].reshape(NW, WS, EMBED).transpose(1, 0, 2).reshape(SEQ, EMBED)
    value_w = value[0].reshape(NW, WS, EMBED).transpose(1, 0, 2).reshape(SEQ, EMBED)

    qhT = _proj_kT(query[0], Wq, bq)[0]
    kTf, kbT = _proj_kT(key_w, Wk, bk)
    vbw = _proj_h(value_w, Wv, bv, BF)

    ensT = _ens(kTf)

    attn3 = _attention(temp.reshape(1, 1), qhT, ensT, kbT, vbw)
    wob = Wo.reshape(EMBED, HEADS, HD).transpose(1, 0, 2).astype(BF)
    out = _proj_out(attn3, wob, bo)
    return out[None]


# speedup vs baseline: 1.0251x; 1.0251x over previous
import jax
import jax.numpy as jnp
from jax.experimental import pallas as pl

EMBED = 768
HEADS = 12
HD = EMBED // HEADS
WS = 16
SEQ = 2048
NW = SEQ // WS
TOPK = 4
SCALE = HD ** -0.5
BQ = 1024
BF = jnp.bfloat16
F32 = jnp.float32


def _proj_h_kernel(out_dtype, x_ref, w_ref, b_ref, o_ref):
    o = jax.lax.dot_general(x_ref[...].astype(BF), w_ref[0].astype(BF),
                            (((1,), (1,)), ((), ())),
                            preferred_element_type=F32) + b_ref[0]
    o_ref[0] = o.astype(out_dtype)


def _proj_h(x, w, b, out_dtype):
    import functools
    return pl.pallas_call(
        functools.partial(_proj_h_kernel, out_dtype),
        grid=(SEQ // BQ, HEADS),
        in_specs=[
            pl.BlockSpec((BQ, EMBED), lambda i, h: (i, 0)),
            pl.BlockSpec((1, HD, EMBED), lambda i, h: (h, 0, 0)),
            pl.BlockSpec((1, 1, HD), lambda i, h: (h, 0, 0)),
        ],
        out_specs=pl.BlockSpec((1, BQ, HD), lambda i, h: (h, i, 0)),
        out_shape=jax.ShapeDtypeStruct((HEADS, SEQ, HD), out_dtype),
    )(x, w.reshape(HEADS, HD, EMBED), b.reshape(HEADS, 1, HD))


def _proj_kT_kernel(x_ref, w_ref, b_ref, of_ref, ob_ref):
    o = jax.lax.dot_general(w_ref[0].astype(BF), x_ref[...].astype(BF),
                            (((1,), (1,)), ((), ())),
                            preferred_element_type=F32) + b_ref[0]
    of_ref[0] = o
    ob_ref[0] = o.astype(BF)


def _proj_kT(x, w, b):
    return pl.pallas_call(
        _proj_kT_kernel,
        grid=(SEQ // BQ, HEADS),
        in_specs=[
            pl.BlockSpec((BQ, EMBED), lambda i, h: (i, 0)),
            pl.BlockSpec((1, HD, EMBED), lambda i, h: (h, 0, 0)),
            pl.BlockSpec((1, HD, 1), lambda i, h: (h, 0, 0)),
        ],
        out_specs=[
            pl.BlockSpec((1, HD, BQ), lambda i, h: (h, 0, i)),
            pl.BlockSpec((1, HD, BQ), lambda i, h: (h, 0, i)),
        ],
        out_shape=[
            jax.ShapeDtypeStruct((HEADS, HD, SEQ), F32),
            jax.ShapeDtypeStruct((HEADS, HD, SEQ), BF),
        ],
    )(x, w.reshape(HEADS, HD, EMBED), b.reshape(HEADS, HD, 1))


def _ens_kernel(k_ref, o_ref):
    kT = k_ref[0]
    nrm2 = jnp.sum(kT * kT, axis=0, keepdims=True)
    knT = kT / jnp.maximum(jnp.sqrt(nrm2), 1e-12)
    knb = knT.astype(BF).astype(F32)
    acc = jnp.zeros((HD * HD, NW), F32)
    for w in range(WS):
        a = knb[:, w * NW:(w + 1) * NW]
        rep = jnp.broadcast_to(a[:, None, :], (HD, HD, NW)).reshape(HD * HD, NW)
        til = jnp.broadcast_to(a[None, :, :], (HD, HD, NW)).reshape(HD * HD, NW)
        acc = acc + rep * til
    o_ref[0] = (acc * (1.0 / WS)).astype(BF)


def _ens(kT):
    return pl.pallas_call(
        _ens_kernel,
        grid=(HEADS,),
        in_specs=[pl.BlockSpec((1, HD, SEQ), lambda h: (h, 0, 0))],
        out_specs=pl.BlockSpec((1, HD * HD, NW), lambda h: (h, 0, 0)),
        out_shape=jax.ShapeDtypeStruct((HEADS, HD * HD, NW), BF),
    )(kT)


def _attn_kernel(t_ref, qt_ref, e_ref, kt_ref, v_ref, o_ref):
    qT = qt_ref[0]
    ensT = e_ref[0]
    kbT = kt_ref[0]
    vb = v_ref[0]
    t = jnp.maximum(t_ref[0, 0], 0.1)

    qn2 = jnp.sum(qT * qT, axis=0, keepdims=True)
    qnT = qT / jnp.maximum(jnp.sqrt(qn2), 1e-12)
    rep = jnp.broadcast_to(qnT[:, None, :], (HD, HD, BQ)).reshape(HD * HD, BQ)
    til = jnp.broadcast_to(qnT[None, :, :], (HD, HD, BQ)).reshape(HD * HD, BQ)
    qdmTb = (rep * til).astype(BF)
    score = jax.lax.dot_general(qdmTb, ensT, (((0,), (0,)), ((), ())),
                                preferred_element_type=F32) / t

    idx = jax.lax.broadcasted_iota(jnp.int32, (BQ, NW), 1)
    vals = []
    hots = []
    work = score
    for _ in range(TOPK):
        m = jnp.max(work, axis=1, keepdims=True)
        am = jnp.min(jnp.where(work == m, idx, NW), axis=1, keepdims=True)
        hot = idx == am
        vals.append(m)
        hots.append(hot)
        work = jnp.where(hot, jnp.float32(-1e30), work)
    es = [jnp.exp(v - vals[0]) for v in vals]
    z = es[0] + es[1] + es[2] + es[3]
    c = (
        es[0] * hots[0].astype(F32)
        + es[1] * hots[1].astype(F32)
        + es[2] * hots[2].astype(F32)
        + es[3] * hots[3].astype(F32)
    ) / z

    sf = jax.lax.dot_general(qT.astype(BF), kbT, (((0,), (0,)), ((), ())),
                             preferred_element_type=F32) * SCALE
    p3 = sf.reshape(BQ, WS, NW)
    pm = jnp.max(p3, axis=1, keepdims=True)
    e = jnp.exp(p3 - pm)
    se = jnp.sum(e, axis=1, keepdims=True)
    fa = e / se

    wts = fa.astype(BF).astype(F32) * c.astype(BF).astype(F32)[:, None, :]
    wb = wts.reshape(BQ, SEQ).astype(BF)
    o_ref[0] = jnp.dot(wb, vb, preferred_element_type=F32)


def _attention(temp2d, qhT, ensT, kbT, vbw):
    return pl.pallas_call(
        _attn_kernel,
        grid=(HEADS, SEQ // BQ),
        in_specs=[
            pl.BlockSpec((1, 1), lambda h, i: (0, 0)),
            pl.BlockSpec((1, HD, BQ), lambda h, i: (h, 0, i)),
            pl.BlockSpec((1, HD * HD, NW), lambda h, i: (h, 0, 0)),
            pl.BlockSpec((1, HD, SEQ), lambda h, i: (h, 0, 0)),
            pl.BlockSpec((1, SEQ, HD), lambda h, i: (h, 0, 0)),
        ],
        out_specs=pl.BlockSpec((1, BQ, HD), lambda h, i: (h, i, 0)),
        out_shape=jax.ShapeDtypeStruct((HEADS, SEQ, HD), F32),
    )(temp2d, qhT, ensT, kbT, vbw)


def _proj_out_kernel(x_ref, w_ref, b_ref, o_ref):
    acc = b_ref[...]
    for h in range(HEADS):
        acc = acc + jax.lax.dot_general(x_ref[h].astype(BF), w_ref[h],
                                        (((1,), (1,)), ((), ())),
                                        preferred_element_type=F32)
    o_ref[...] = acc


def _proj_out(attn3, wob, b):
    return pl.pallas_call(
        _proj_out_kernel,
        grid=(SEQ // BQ,),
        in_specs=[
            pl.BlockSpec((HEADS, BQ, HD), lambda i: (0, i, 0)),
            pl.BlockSpec((HEADS, EMBED, HD), lambda i: (0, 0, 0)),
            pl.BlockSpec((1, EMBED), lambda i: (0, 0)),
        ],
        out_specs=pl.BlockSpec((BQ, EMBED), lambda i: (i, 0)),
        out_shape=jax.ShapeDtypeStruct((SEQ, EMBED), F32),
    )(attn3, wob, b.reshape(1, EMBED))


@jax.jit
def kernel(query, key, value, Wq, bq, Wk, bk, Wv, bv, Wo, bo, temp):
    key_w = key[0].reshape(NW, WS, EMBED).transpose(1, 0, 2).reshape(SEQ, EMBED)
    value_w = value[0].reshape(NW, WS, EMBED).transpose(1, 0, 2).reshape(SEQ, EMBED)

    qhT = _proj_kT(query[0], Wq, bq)[0]
    kTf, kbT = _proj_kT(key_w, Wk, bk)
    vbw = _proj_h(value_w, Wv, bv, BF)

    ensT = _ens(kTf)

    attn3 = _attention(temp.reshape(1, 1), qhT, ensT, kbT, vbw)
    wob = Wo.reshape(EMBED, HEADS, HD).transpose(1, 0, 2).astype(BF)
    out = _proj_out(attn3, wob, bo)
    return out[None]
